# Initial kernel scaffold; baseline (speedup 1.0000x reference)
import jax
import jax.numpy as jnp
from jax.experimental import pallas as pl


def kernel(z, in_proj_v, in_proj_g, in_proj_b, out_proj_v, out_proj_g, out_proj_b, codebook):
    raise NotImplementedError("write your pallas kernel here")


# speedup vs baseline: 1.3744x; 1.3744x over previous
import functools

import jax
import jax.numpy as jnp
from jax import lax
from jax.experimental import pallas as pl
from jax.experimental.pallas import tpu as pltpu
from jax.experimental.pallas import tpu_sc as plsc

B, D, T = 8, 1024, 2048
K, Dc = 8192, 256
N = B * T
MT = 256
KT = 512
EPS = 1e-12


def _vq_stage1(z, w_in, b_in, codebook):
    n_tc = T // MT
    n_kc = K // KT

    def body(z_ref, win_ref, bin_ref, cb_ref, ze_ref, idx_ref, cbn_ref, cvec_ref):
        b = pl.program_id(0)
        tc = pl.program_id(1)

        @pl.when(jnp.logical_and(b == 0, tc == 0))
        def _init():
            for j in range(n_kc):
                rows = cb_ref[j * KT:(j + 1) * KT, :]
                n = jnp.sqrt(jnp.sum(rows * rows, axis=1, keepdims=True))
                rn = rows / jnp.maximum(n, EPS)
                cbn_ref[j * KT:(j + 1) * KT, :] = rn
                cvec_ref[j * KT:(j + 1) * KT, :] = jnp.sum(
                    rn * rn, axis=1, keepdims=True)

        zblk = z_ref[0]
        ze = jnp.dot(win_ref[...], zblk,
                     precision=lax.Precision.HIGHEST) + bin_ref[...]
        nsq = jnp.sum(ze * ze, axis=0, keepdims=True)
        zen = ze / jnp.maximum(jnp.sqrt(nsq), EPS)
        a = jnp.sum(zen * zen, axis=0, keepdims=True)

        best_val = jnp.full((1, MT), -jnp.inf, jnp.float32)
        best_arg = jnp.zeros((1, MT), jnp.int32)
        for i in range(n_kc):
            cc = cbn_ref[i * KT:(i + 1) * KT, :]
            cv = cvec_ref[i * KT:(i + 1) * KT, :]
            dots = jnp.dot(cc, zen)
            dist = (a - 2.0 * dots) + cv
            neg = -dist
            lm = jnp.max(neg, axis=0, keepdims=True)
            la = jnp.argmax(neg, axis=0, keepdims=True).astype(jnp.int32) + i * KT
            better = lm > best_val
            best_val = jnp.where(better, lm, best_val)
            best_arg = jnp.where(better, la, best_arg)
        idx_ref[0] = best_arg
        ze_ref[...] = ze.T

    return pl.pallas_call(
        body,
        grid=(B, n_tc),
        in_specs=[
            pl.BlockSpec((1, D, MT), lambda b, t: (b, 0, t)),
            pl.BlockSpec((Dc, D), lambda b, t: (0, 0)),
            pl.BlockSpec((Dc, 1), lambda b, t: (0, 0)),
            pl.BlockSpec((K, Dc), lambda b, t: (0, 0)),
        ],
        out_specs=[
            pl.BlockSpec((MT, Dc), lambda b, t: (b * (T // MT) + t, 0)),
            pl.BlockSpec((1, 1, MT), lambda b, t: (b, 0, t)),
        ],
        out_shape=[
            jax.ShapeDtypeStruct((N, Dc), jnp.float32),
            jax.ShapeDtypeStruct((B, 1, T), jnp.int32),
        ],
        scratch_shapes=[
            pltpu.VMEM((K, Dc), jnp.float32),
            pltpu.VMEM((K, 1), jnp.float32),
        ],
    )(z, w_in, b_in, codebook)


def _gather(codebook, idx_flat):
    info = plsc.get_sparse_core_info()
    nc, ns = info.num_cores, info.num_subcores
    nw = nc * ns
    bpw = N // nw
    ch = 128
    mesh = plsc.VectorSubcoreMesh(core_axis_name="c", subcore_axis_name="s")

    @functools.partial(
        pl.kernel,
        mesh=mesh,
        out_type=jax.ShapeDtypeStruct((N, Dc), jnp.float32),
        scratch_types=[
            pltpu.VMEM((ch,), jnp.int32),
            pltpu.VMEM((ch, Dc), jnp.float32),
            pltpu.SemaphoreType.DMA,
        ],
    )
    def gk(idx_hbm, table_hbm, out_hbm, idx_v, rows_v, sem):
        wid = lax.axis_index("s") * nc + lax.axis_index("c")
        base = wid * bpw
        for c in range(bpw // ch):
            off = base + c * ch
            pltpu.sync_copy(idx_hbm.at[pl.ds(off, ch)], idx_v)
            pltpu.async_copy(table_hbm.at[idx_v], rows_v, sem).wait()
            pltpu.sync_copy(rows_v, out_hbm.at[pl.ds(off, ch)])

    return gk(idx_flat, codebook)


def _stage3(zq_tok, ze_tok, w_out, b_out):
    n_tc = T // MT

    def body(zq_ref, ze_ref, wout_ref, bout_ref, out_ref, loss_ref):
        b = pl.program_id(0)
        tc = pl.program_id(1)
        zq = zq_ref[...]
        o = lax.dot_general(wout_ref[...], zq, (((1,), (1,)), ((), ())),
                            preferred_element_type=jnp.float32)
        out_ref[0] = o + bout_ref[...]
        diff = ze_ref[...] - zq
        s = jnp.sum(diff * diff)

        @pl.when(jnp.logical_and(b == 0, tc == 0))
        def _init():
            for i in range(B):
                loss_ref[i, 0] = 0.0

        loss_ref[b, 0] += s

    return pl.pallas_call(
        body,
        grid=(B, n_tc),
        in_specs=[
            pl.BlockSpec((MT, Dc), lambda b, t: (b * (T // MT) + t, 0)),
            pl.BlockSpec((MT, Dc), lambda b, t: (b * (T // MT) + t, 0)),
            pl.BlockSpec((D, Dc), lambda b, t: (0, 0)),
            pl.BlockSpec((D, 1), lambda b, t: (0, 0)),
        ],
        out_specs=[
            pl.BlockSpec((1, D, MT), lambda b, t: (b, 0, t)),
            pl.BlockSpec((B, 1), lambda b, t: (0, 0), memory_space=pltpu.SMEM),
        ],
        out_shape=[
            jax.ShapeDtypeStruct((B, D, T), jnp.float32),
            jax.ShapeDtypeStruct((B, 1), jnp.float32),
        ],
    )(zq_tok, ze_tok, w_out, b_out)


def kernel(z, in_proj_v, in_proj_g, in_proj_b, out_proj_v, out_proj_g, out_proj_b, codebook):
    n_in = jnp.sqrt(jnp.sum(in_proj_v * in_proj_v, axis=1, keepdims=True))
    w_in = (in_proj_g[:, None] * in_proj_v) / n_in
    n_out = jnp.sqrt(jnp.sum(out_proj_v * out_proj_v, axis=1, keepdims=True))
    w_out = (out_proj_g[:, None] * out_proj_v) / n_out

    ze_tok, idx_col = _vq_stage1(z, w_in, in_proj_b[:, None], codebook)
    idx = idx_col.reshape(B, T)
    zq_tok = _gather(codebook, idx_col.reshape(N))
    out, loss_sum = _stage3(zq_tok, ze_tok, w_out, out_proj_b[:, None])
    loss = loss_sum[:, 0] / float(Dc * T)
    return (out, loss, loss, idx)
